# baseline (speedup 1.0000x reference)
import functools

import jax
import jax.numpy as jnp
from jax import lax
from jax.experimental import pallas as pl
from jax.experimental.pallas import tpu as pltpu
from jax.experimental.pallas import tpu_sc as plsc

N = 10000
E = 320000
D = 128

_NB1 = 10
_BN = N // _NB1
_NBE = 160
_BE = E // _NBE


def _gelu(x):
    return 0.5 * x * (1.0 + jax.lax.erf(x * 0.7071067811865476))


def _tc1_body(x_ref, wsp_ref, wq_ref, wp_ref, wp2_ref, bp_ref, bp2_ref,
              s_ref, t_ref, h_ref):
    x = x_ref[...]
    s_ref[:, :D] = x
    s_ref[:, D:] = jnp.dot(x, wsp_ref[...], preferred_element_type=jnp.float32)
    t_ref[:, :D] = x
    t_ref[:, D:] = jnp.dot(x, wq_ref[...], preferred_element_type=jnp.float32)
    h_ref[:, :D] = _gelu(jnp.dot(x, wp_ref[...], preferred_element_type=jnp.float32)
                         + bp_ref[...])
    h_ref[:, D:] = _gelu(jnp.dot(x, wp2_ref[...], preferred_element_type=jnp.float32)
                         + bp2_ref[...])


def _tc1(x, wsp, wq, wp, wp2, bp, bp2):
    full = lambda shape: pl.BlockSpec(shape, lambda i: (0,) * len(shape))
    return pl.pallas_call(
        _tc1_body,
        grid=(_NB1,),
        in_specs=[
            pl.BlockSpec((_BN, D), lambda i: (i, 0)),
            full((D, D)), full((D, D)), full((D, D)), full((D, D)),
            full((1, D)), full((1, D)),
        ],
        out_specs=[
            pl.BlockSpec((_BN, 2 * D), lambda i: (i, 0)),
            pl.BlockSpec((_BN, 2 * D), lambda i: (i, 0)),
            pl.BlockSpec((_BN, 2 * D), lambda i: (i, 0)),
        ],
        out_shape=[
            jax.ShapeDtypeStruct((N, 2 * D), jnp.float32),
            jax.ShapeDtypeStruct((N, 2 * D), jnp.float32),
            jax.ShapeDtypeStruct((N, 2 * D), jnp.float32),
        ],
    )(x, wsp, wq, wp, wp2, bp, bp2)


_NC, _NS = 2, 16
_NW = _NC * _NS
_GB = 128
_NCHUNK = E // _GB
_CPW = -(-_NCHUNK // _NW)


def _sc1(s_tab, t_tab, src, dst):
    mesh = plsc.VectorSubcoreMesh(core_axis_name="c", subcore_axis_name="s",
                                  num_cores=_NC, num_subcores=_NS)

    @functools.partial(
        pl.kernel,
        out_type=[jax.ShapeDtypeStruct((E, 2 * D), jnp.float32),
                  jax.ShapeDtypeStruct((E, 2 * D), jnp.float32)],
        mesh=mesh,
        scratch_types=[pltpu.VMEM((_GB,), jnp.int32),
                       pltpu.VMEM((_GB, 2 * D), jnp.float32),
                       pltpu.SemaphoreType.DMA],
    )
    def k(s_hbm, t_hbm, src_hbm, dst_hbm, xs_hbm, xd_hbm, idx_v, rows_v, sem):
        wid = lax.axis_index("s") * _NC + lax.axis_index("c")

        def chunk_body(j, carry):
            c = wid + _NW * j

            @pl.when(c < _NCHUNK)
            def _():
                base = c * _GB
                pltpu.sync_copy(src_hbm.at[pl.ds(base, _GB)], idx_v)
                pltpu.async_copy(s_hbm.at[idx_v], rows_v, sem).wait()
                pltpu.sync_copy(rows_v, xs_hbm.at[pl.ds(base, _GB)])
                pltpu.sync_copy(dst_hbm.at[pl.ds(base, _GB)], idx_v)
                pltpu.async_copy(t_hbm.at[idx_v], rows_v, sem).wait()
                pltpu.sync_copy(rows_v, xd_hbm.at[pl.ds(base, _GB)])

            return carry

        lax.fori_loop(0, _CPW, chunk_body, 0)

    return k(s_tab, t_tab, src, dst)


def _tc2_body(xs_ref, xd_ref, wm_ref, ball_ref, wout_ref, e_ref):
    xs = xs_ref[...]
    xd = xd_ref[...]
    g = xs[:, :D] * xd[:, :D]
    z = (jnp.dot(g, wm_ref[...], preferred_element_type=jnp.float32)
         + xs[:, D:] + xd[:, D:] + ball_ref[...])
    ge = _gelu(z)
    s = jnp.sum(ge * wout_ref[0, :D], axis=1) + wout_ref[0, D]
    e_ref[0, 0, :] = jnp.where(s > 0, s, 0.2 * s)


def _tc2(xs, xd, wm, ball, woutb):
    full = lambda shape: pl.BlockSpec(shape, lambda i: (0,) * len(shape))
    e3 = pl.pallas_call(
        _tc2_body,
        grid=(_NBE,),
        in_specs=[
            pl.BlockSpec((_BE, 2 * D), lambda i: (i, 0)),
            pl.BlockSpec((_BE, 2 * D), lambda i: (i, 0)),
            full((D, D)), full((1, D)), full((1, D + 1)),
        ],
        out_specs=pl.BlockSpec((1, 1, _BE), lambda i: (i, 0, 0)),
        out_shape=jax.ShapeDtypeStruct((_NBE, 1, _BE), jnp.float32),
    )(xs, xd, wm, ball, woutb)
    return e3.reshape(E)


_TPB = 313
_ACC = 320
_TRASH = 313
_SCC = 2000
_NSCC = E // _SCC
_B2 = 128
_GRP = 5
_CAP = 224


def _sc2(src, dst, e, hpack):
    mesh = plsc.VectorSubcoreMesh(core_axis_name="c", subcore_axis_name="s",
                                  num_cores=_NC, num_subcores=_NS)

    @functools.partial(
        pl.kernel,
        out_type=[jax.ShapeDtypeStruct((_NW, _ACC, D), jnp.float32),
                  jax.ShapeDtypeStruct((_NW, _ACC, D), jnp.float32),
                  jax.ShapeDtypeStruct((_NW, _ACC), jnp.float32)],
        mesh=mesh,
        compiler_params=pltpu.CompilerParams(needs_layout_passes=False),
        scratch_types=[
            pltpu.VMEM((_SCC,), jnp.int32),
            pltpu.VMEM((_SCC,), jnp.int32),
            pltpu.VMEM((_SCC,), jnp.int32),
            pltpu.VMEM((_SCC,), jnp.int32),
            pltpu.VMEM((_SCC,), jnp.float32),
            pltpu.VMEM((_SCC,), jnp.float32),
            pltpu.VMEM((_CAP,), jnp.int32),
            pltpu.VMEM((_CAP,), jnp.int32),
            pltpu.VMEM((_CAP,), jnp.float32),
            pltpu.VMEM((_B2,), jnp.int32),
            pltpu.VMEM((_B2,), jnp.float32),
            pltpu.VMEM((_B2, 2 * D), jnp.float32),
            pltpu.VMEM((_ACC, D), jnp.float32),
            pltpu.VMEM((_ACC, D), jnp.float32),
            pltpu.VMEM((_ACC,), jnp.float32),
            pltpu.SemaphoreType.DMA,
            pltpu.SemaphoreType.DMA,
            pltpu.SemaphoreType.DMA,
        ],
    )
    def k(src_hbm, dst_hbm, e_hbm, h_hbm,
          mx_hbm, sm_hbm, cnt_hbm,
          dbuf0, dbuf1, sbuf0, sbuf1, ebuf0, ebuf1, csrc, cdl, ce,
          sdl, sev, grows, mxacc, smacc, cntacc, semA, semB, semg):
        wid = lax.axis_index("s") * _NC + lax.axis_index("c")
        lo = wid * _TPB

        neg = jnp.full((16,), -jnp.inf, jnp.float32)
        zero = jnp.zeros((16,), jnp.float32)
        ones_i = jnp.ones((16,), jnp.int32)
        zeros_i = jnp.zeros((16,), jnp.int32)
        ones_f = jnp.ones((16,), jnp.float32)

        def init_body(r, carry):
            for kk in range(D // 16):
                sl = pl.ds(kk * 16, 16)
                mxacc[r, sl] = neg
                smacc[r, sl] = zero
            return carry

        lax.fori_loop(0, _ACC, init_body, 0)
        for kk in range(_ACC // 16):
            cntacc[pl.ds(kk * 16, 16)] = zero

        def issue(ci, db, sb, eb, sem):
            base = ci * _SCC
            pltpu.async_copy(dst_hbm.at[pl.ds(base, _SCC)], db, sem)
            pltpu.async_copy(src_hbm.at[pl.ds(base, _SCC)], sb, sem)
            pltpu.async_copy(e_hbm.at[pl.ds(base, _SCC)], eb, sem)

        def drain(db, sb, eb, sem):
            pltpu.make_async_copy(dst_hbm.at[pl.ds(0, _SCC)], db, sem).wait()
            pltpu.make_async_copy(src_hbm.at[pl.ds(0, _SCC)], sb, sem).wait()
            pltpu.make_async_copy(e_hbm.at[pl.ds(0, _SCC)], eb, sem).wait()

        def fire_gather():
            pltpu.async_copy(h_hbm.at[csrc.at[pl.ds(0, _B2)]], grows, semg)

        def wait_gather():
            pltpu.make_async_copy(h_hbm.at[pl.ds(0, _B2)], grows, semg).wait()

        def accumulate(dlref, evref):
            def edge_grp(g, carry):
                base = g * 16
                dlv = dlref[pl.ds(base, 16)]
                evv_all = evref[pl.ds(base, 16)]
                for j in range(16):
                    i = base + j
                    dl = dlv[j]
                    evv = jnp.full((16,), evv_all[j], jnp.float32)
                    for kk in range(D // 16):
                        sl = pl.ds(kk * 16, 16)
                        sl2 = pl.ds(D + kk * 16, 16)
                        mxacc[dl, sl] = jnp.maximum(mxacc[dl, sl],
                                                    evv * grows[i, sl])
                        plsc.addupdate(smacc.at[dl, sl], evv * grows[i, sl2])
                return carry

            lax.fori_loop(0, _B2 // 16, edge_grp, 0)

        def flush(pend):
            @pl.when(pend > 0)
            def _():
                wait_gather()
                accumulate(sdl, sev)

            for j in range(_B2 // 16):
                sl = pl.ds(j * 16, 16)
                sdl[sl] = cdl[sl]
                sev[sl] = ce[sl]
            fire_gather()

        def scan_chunk(db, sb, eb, ptr):
            def group_body(g, carry):
                ptr, pend = carry
                base = g * (_GRP * 16)
                uu, mm, pp = [], [], []
                for v in range(_GRP):
                    sl = pl.ds(base + v * 16, 16)
                    u = db[sl] - lo
                    m = (u >= 0) & (u < _TPB)
                    sel = lax.select(m, ones_i, zeros_i)
                    uu.append(u)
                    mm.append(m)
                    pp.append(plsc.cumsum(sel)[15])
                    plsc.addupdate_scatter(cntacc, [u], ones_f, mask=m)
                for v in range(_GRP):
                    sl = pl.ds(base + v * 16, 16)
                    psl = pl.ds(ptr, 16)
                    plsc.store_compressed(csrc.at[psl], sb[sl], mask=mm[v])
                    plsc.store_compressed(cdl.at[psl], uu[v], mask=mm[v])
                    plsc.store_compressed(ce.at[psl], eb[sl], mask=mm[v])
                    ptr = ptr + pp[v]
                do_flush = ptr >= _B2

                @pl.when(do_flush)
                def _():
                    flush(pend)
                    for j in range((_CAP - _B2) // 16):
                        s1 = pl.ds(j * 16, 16)
                        s2 = pl.ds(_B2 + j * 16, 16)
                        csrc[s1] = csrc[s2]
                        cdl[s1] = cdl[s2]
                        ce[s1] = ce[s2]

                return (jnp.where(do_flush, ptr - _B2, ptr),
                        jnp.where(do_flush, 1, pend))

            return lax.fori_loop(0, _SCC // (16 * _GRP), group_body, ptr)

        issue(0, dbuf0, sbuf0, ebuf0, semA)

        def pair_body(p, carry):
            ci0 = 2 * p
            issue(ci0 + 1, dbuf1, sbuf1, ebuf1, semB)
            drain(dbuf0, sbuf0, ebuf0, semA)
            carry = scan_chunk(dbuf0, sbuf0, ebuf0, carry)

            @pl.when(ci0 + 2 < _NSCC)
            def _():
                issue(ci0 + 2, dbuf0, sbuf0, ebuf0, semA)

            drain(dbuf1, sbuf1, ebuf1, semB)
            return scan_chunk(dbuf1, sbuf1, ebuf1, carry)

        ptr, pend = lax.fori_loop(0, _NSCC // 2, pair_body,
                                  (jnp.int32(0), jnp.int32(0)))

        @pl.when(pend > 0)
        def _():
            wait_gather()
            accumulate(sdl, sev)

        lane = lax.iota(jnp.int32, 16)
        for j in range(_B2 // 16):
            sl = pl.ds(j * 16, 16)
            keep = (lane + j * 16) < ptr
            cdl[sl] = jnp.where(keep, cdl[sl], _TRASH)
            ce[sl] = jnp.where(keep, ce[sl], 0.0)
            csrc[sl] = jnp.where(keep, csrc[sl], 0)
        fire_gather()
        wait_gather()
        accumulate(cdl, ce)

        pltpu.sync_copy(mxacc, mx_hbm.at[wid])
        pltpu.sync_copy(smacc, sm_hbm.at[wid])
        pltpu.sync_copy(cntacc, cnt_hbm.at[wid])

    return k(src, dst, e, hpack)


def _tc3_body(x_ref, mx_ref, sm_ref, cnt_ref,
              wself_ref, wneigh_ref, wneigh2_ref, wm0_ref, wm1_ref,
              b0_ref, bm0_ref, bm1_ref, out_ref):
    x = x_ref[...]
    mx = mx_ref[...]
    neigh = jnp.where(jnp.isfinite(mx), mx, 0.0)
    neigh2 = sm_ref[...] / jnp.maximum(cnt_ref[...], 1.0)
    rst = (jnp.dot(x, wself_ref[...], preferred_element_type=jnp.float32)
           + jnp.dot(neigh, wneigh_ref[...], preferred_element_type=jnp.float32)
           + jnp.dot(neigh2, wneigh2_ref[...], preferred_element_type=jnp.float32)
           + b0_ref[...])
    rst = rst + jnp.dot(_gelu(rst), wm0_ref[...],
                        preferred_element_type=jnp.float32) + bm0_ref[...]
    rst = rst + jnp.dot(_gelu(rst), wm1_ref[...],
                        preferred_element_type=jnp.float32) + bm1_ref[...]
    out_ref[...] = rst


def _tc3(x, mx, sm, cnt, wself, wneigh, wneigh2, wm0, wm1, b0, bm0, bm1):
    full = lambda shape: pl.BlockSpec(shape, lambda i: (0,) * len(shape))
    return pl.pallas_call(
        _tc3_body,
        grid=(_NB1,),
        in_specs=[
            pl.BlockSpec((_BN, D), lambda i: (i, 0)),
            pl.BlockSpec((_BN, D), lambda i: (i, 0)),
            pl.BlockSpec((_BN, D), lambda i: (i, 0)),
            pl.BlockSpec((_BN, 1), lambda i: (i, 0)),
            full((D, D)), full((D, D)), full((D, D)), full((D, D)), full((D, D)),
            full((1, D)), full((1, D)), full((1, D)),
        ],
        out_specs=pl.BlockSpec((_BN, D), lambda i: (i, 0)),
        out_shape=jax.ShapeDtypeStruct((N, D), jnp.float32),
    )(x, mx, sm, cnt, wself, wneigh, wneigh2, wm0, wm1, b0, bm0, bm1)


def kernel(x, edge_index, params):
    src = edge_index[0]
    dst = edge_index[1]

    wsp = (params['W_sub'] + params['W_src']).T
    wq = (params['W_dst'] - params['W_sub']).T
    wp = params['W_pool'].T
    wp2 = params['W_pool2'].T
    bp = params['b_pool'].reshape(1, D)
    bp2 = params['b_pool2'].reshape(1, D)
    s_tab, t_tab, hpack = _tc1(x, wsp, wq, wp, wp2, bp, bp2)

    xs, xd = _sc1(s_tab, t_tab, src, dst)

    wm = params['W_mul'].T
    ball = (params['b_sub'] + params['b_src'] + params['b_dst']
            + params['b_mul']).reshape(1, D)
    woutb = jnp.concatenate([params['W_out'][0], params['b_out']]).reshape(1, D + 1)
    e = _tc2(xs, xd, wm, ball, woutb)

    mx_h, sm_h, cnt_h = _sc2(src, dst, e, hpack)
    mx = mx_h[:, :_TPB, :].reshape(_NW * _TPB, D)[:N]
    sm = sm_h[:, :_TPB, :].reshape(_NW * _TPB, D)[:N]
    cnt = cnt_h[:, :_TPB].reshape(_NW * _TPB)[:N].reshape(N, 1)

    return _tc3(x, mx, sm, cnt,
                params['W_self'].T, params['W_neigh'].T, params['W_neigh2'].T,
                params['W_mlp0'].T, params['W_mlp1'].T,
                (params['b_self'] + params['b_neigh']
                 + params['b_neigh2']).reshape(1, D),
                params['b_mlp0'].reshape(1, D),
                params['b_mlp1'].reshape(1, D))


# speedup vs baseline: 1.0580x; 1.0580x over previous
import functools

import jax
import jax.numpy as jnp
from jax import lax
from jax.experimental import pallas as pl
from jax.experimental.pallas import tpu as pltpu
from jax.experimental.pallas import tpu_sc as plsc

N = 10000
E = 320000
D = 128

_NB1 = 10
_BN = N // _NB1
_NBE = 160
_BE = E // _NBE


def _gelu(x):
    return 0.5 * x * (1.0 + jax.lax.erf(x * 0.7071067811865476))


def _tc1_body(x_ref, wsp_ref, wq_ref, wp_ref, wp2_ref, bp_ref, bp2_ref,
              s_ref, t_ref, h_ref):
    x = x_ref[...]
    s_ref[:, :D] = x
    s_ref[:, D:] = jnp.dot(x, wsp_ref[...], preferred_element_type=jnp.float32)
    t_ref[:, :D] = x
    t_ref[:, D:] = jnp.dot(x, wq_ref[...], preferred_element_type=jnp.float32)
    h_ref[:, :D] = _gelu(jnp.dot(x, wp_ref[...], preferred_element_type=jnp.float32)
                         + bp_ref[...])
    h_ref[:, D:] = _gelu(jnp.dot(x, wp2_ref[...], preferred_element_type=jnp.float32)
                         + bp2_ref[...])


def _tc1(x, wsp, wq, wp, wp2, bp, bp2):
    full = lambda shape: pl.BlockSpec(shape, lambda i: (0,) * len(shape))
    return pl.pallas_call(
        _tc1_body,
        grid=(_NB1,),
        in_specs=[
            pl.BlockSpec((_BN, D), lambda i: (i, 0)),
            full((D, D)), full((D, D)), full((D, D)), full((D, D)),
            full((1, D)), full((1, D)),
        ],
        out_specs=[
            pl.BlockSpec((_BN, 2 * D), lambda i: (i, 0)),
            pl.BlockSpec((_BN, 2 * D), lambda i: (i, 0)),
            pl.BlockSpec((_BN, 2 * D), lambda i: (i, 0)),
        ],
        out_shape=[
            jax.ShapeDtypeStruct((N, 2 * D), jnp.float32),
            jax.ShapeDtypeStruct((N, 2 * D), jnp.float32),
            jax.ShapeDtypeStruct((N, 2 * D), jnp.float32),
        ],
    )(x, wsp, wq, wp, wp2, bp, bp2)


_NC, _NS = 2, 16
_NW = _NC * _NS
_GB = 128
_NCHUNK = E // _GB
_CPW = -(-_NCHUNK // _NW)


def _sc1(s_tab, t_tab, src, dst):
    mesh = plsc.VectorSubcoreMesh(core_axis_name="c", subcore_axis_name="s",
                                  num_cores=_NC, num_subcores=_NS)

    @functools.partial(
        pl.kernel,
        out_type=[jax.ShapeDtypeStruct((E, 2 * D), jnp.float32),
                  jax.ShapeDtypeStruct((E, 2 * D), jnp.float32)],
        mesh=mesh,
        scratch_types=[pltpu.VMEM((_GB,), jnp.int32),
                       pltpu.VMEM((_GB,), jnp.int32),
                       pltpu.VMEM((_GB, 2 * D), jnp.float32),
                       pltpu.VMEM((_GB, 2 * D), jnp.float32),
                       pltpu.SemaphoreType.DMA,
                       pltpu.SemaphoreType.DMA,
                       pltpu.SemaphoreType.DMA],
    )
    def k(s_hbm, t_hbm, src_hbm, dst_hbm, xs_hbm, xd_hbm,
          idx_s, idx_d, rows_s, rows_d, semi, semgs, semgd):
        wid = lax.axis_index("s") * _NC + lax.axis_index("c")

        def chunk_body(j, carry):
            c = wid + _NW * j

            @pl.when(c < _NCHUNK)
            def _():
                base = c * _GB
                pltpu.async_copy(src_hbm.at[pl.ds(base, _GB)], idx_s, semi)
                pltpu.async_copy(dst_hbm.at[pl.ds(base, _GB)], idx_d, semi)
                pltpu.make_async_copy(src_hbm.at[pl.ds(0, _GB)], idx_s,
                                      semi).wait()
                pltpu.make_async_copy(src_hbm.at[pl.ds(0, _GB)], idx_d,
                                      semi).wait()
                pltpu.async_copy(s_hbm.at[idx_s], rows_s, semgs)
                pltpu.async_copy(t_hbm.at[idx_d], rows_d, semgd)
                pltpu.make_async_copy(s_hbm.at[pl.ds(0, _GB)], rows_s,
                                      semgs).wait()
                pltpu.sync_copy(rows_s, xs_hbm.at[pl.ds(base, _GB)])
                pltpu.make_async_copy(s_hbm.at[pl.ds(0, _GB)], rows_d,
                                      semgd).wait()
                pltpu.sync_copy(rows_d, xd_hbm.at[pl.ds(base, _GB)])

            return carry

        lax.fori_loop(0, _CPW, chunk_body, 0)

    return k(s_tab, t_tab, src, dst)


def _tc2_body(xs_ref, xd_ref, wm_ref, ball_ref, wout_ref, e_ref):
    xs = xs_ref[...]
    xd = xd_ref[...]
    g = xs[:, :D] * xd[:, :D]
    z = (jnp.dot(g, wm_ref[...], preferred_element_type=jnp.float32)
         + xs[:, D:] + xd[:, D:] + ball_ref[...])
    ge = _gelu(z)
    s = jnp.sum(ge * wout_ref[0, :D], axis=1) + wout_ref[0, D]
    e_ref[0, 0, :] = jnp.where(s > 0, s, 0.2 * s)


def _tc2(xs, xd, wm, ball, woutb):
    full = lambda shape: pl.BlockSpec(shape, lambda i: (0,) * len(shape))
    e3 = pl.pallas_call(
        _tc2_body,
        grid=(_NBE,),
        in_specs=[
            pl.BlockSpec((_BE, 2 * D), lambda i: (i, 0)),
            pl.BlockSpec((_BE, 2 * D), lambda i: (i, 0)),
            full((D, D)), full((1, D)), full((1, D + 1)),
        ],
        out_specs=pl.BlockSpec((1, 1, _BE), lambda i: (i, 0, 0)),
        out_shape=jax.ShapeDtypeStruct((_NBE, 1, _BE), jnp.float32),
    )(xs, xd, wm, ball, woutb)
    return e3.reshape(E)


_TPB = 313
_ACC = 320
_TRASH = 313
_SCC = 2000
_NSCC = E // _SCC
_B2 = 128
_GRP = 5
_CAP = 224


def _sc2(src, dst, e, hpack):
    mesh = plsc.VectorSubcoreMesh(core_axis_name="c", subcore_axis_name="s",
                                  num_cores=_NC, num_subcores=_NS)

    @functools.partial(
        pl.kernel,
        out_type=[jax.ShapeDtypeStruct((_NW, _ACC, D), jnp.float32),
                  jax.ShapeDtypeStruct((_NW, _ACC, D), jnp.float32),
                  jax.ShapeDtypeStruct((_NW, _ACC), jnp.float32)],
        mesh=mesh,
        compiler_params=pltpu.CompilerParams(needs_layout_passes=False),
        scratch_types=[
            pltpu.VMEM((_SCC,), jnp.int32),
            pltpu.VMEM((_SCC,), jnp.int32),
            pltpu.VMEM((_SCC,), jnp.int32),
            pltpu.VMEM((_SCC,), jnp.int32),
            pltpu.VMEM((_SCC,), jnp.float32),
            pltpu.VMEM((_SCC,), jnp.float32),
            pltpu.VMEM((_CAP,), jnp.int32),
            pltpu.VMEM((_CAP,), jnp.int32),
            pltpu.VMEM((_CAP,), jnp.float32),
            pltpu.VMEM((_B2,), jnp.int32),
            pltpu.VMEM((_B2,), jnp.float32),
            pltpu.VMEM((_B2, 2 * D), jnp.float32),
            pltpu.VMEM((_ACC, D), jnp.float32),
            pltpu.VMEM((_ACC, D), jnp.float32),
            pltpu.VMEM((_ACC,), jnp.float32),
            pltpu.SemaphoreType.DMA,
            pltpu.SemaphoreType.DMA,
            pltpu.SemaphoreType.DMA,
        ],
    )
    def k(src_hbm, dst_hbm, e_hbm, h_hbm,
          mx_hbm, sm_hbm, cnt_hbm,
          dbuf0, dbuf1, sbuf0, sbuf1, ebuf0, ebuf1, csrc, cdl, ce,
          sdl, sev, grows, mxacc, smacc, cntacc, semA, semB, semg):
        wid = lax.axis_index("s") * _NC + lax.axis_index("c")
        lo = wid * _TPB

        neg = jnp.full((16,), -jnp.inf, jnp.float32)
        zero = jnp.zeros((16,), jnp.float32)
        ones_i = jnp.ones((16,), jnp.int32)
        zeros_i = jnp.zeros((16,), jnp.int32)
        ones_f = jnp.ones((16,), jnp.float32)

        def init_body(r, carry):
            for kk in range(D // 16):
                sl = pl.ds(kk * 16, 16)
                mxacc[r, sl] = neg
                smacc[r, sl] = zero
            return carry

        lax.fori_loop(0, _ACC, init_body, 0)
        for kk in range(_ACC // 16):
            cntacc[pl.ds(kk * 16, 16)] = zero

        def issue(ci, db, sb, eb, sem):
            base = ci * _SCC
            pltpu.async_copy(dst_hbm.at[pl.ds(base, _SCC)], db, sem)
            pltpu.async_copy(src_hbm.at[pl.ds(base, _SCC)], sb, sem)
            pltpu.async_copy(e_hbm.at[pl.ds(base, _SCC)], eb, sem)

        def drain(db, sb, eb, sem):
            pltpu.make_async_copy(dst_hbm.at[pl.ds(0, _SCC)], db, sem).wait()
            pltpu.make_async_copy(src_hbm.at[pl.ds(0, _SCC)], sb, sem).wait()
            pltpu.make_async_copy(e_hbm.at[pl.ds(0, _SCC)], eb, sem).wait()

        def fire_gather():
            pltpu.async_copy(h_hbm.at[csrc.at[pl.ds(0, _B2)]], grows, semg)

        def wait_gather():
            pltpu.make_async_copy(h_hbm.at[pl.ds(0, _B2)], grows, semg).wait()

        def accumulate(dlref, evref):
            def edge_grp(g, carry):
                base = g * 16
                dlv = dlref[pl.ds(base, 16)]
                evv_all = evref[pl.ds(base, 16)]
                for j in range(16):
                    i = base + j
                    dl = dlv[j]
                    evv = jnp.full((16,), evv_all[j], jnp.float32)
                    for kk in range(D // 16):
                        sl = pl.ds(kk * 16, 16)
                        sl2 = pl.ds(D + kk * 16, 16)
                        mxacc[dl, sl] = jnp.maximum(mxacc[dl, sl],
                                                    evv * grows[i, sl])
                        plsc.addupdate(smacc.at[dl, sl], evv * grows[i, sl2])
                return carry

            lax.fori_loop(0, _B2 // 16, edge_grp, 0)

        def flush(pend):
            @pl.when(pend > 0)
            def _():
                wait_gather()
                accumulate(sdl, sev)

            for j in range(_B2 // 16):
                sl = pl.ds(j * 16, 16)
                sdl[sl] = cdl[sl]
                sev[sl] = ce[sl]
            fire_gather()

        def scan_chunk(db, sb, eb, ptr):
            def group_body(g, carry):
                ptr, pend = carry
                base = g * (_GRP * 16)
                uu, mm, pp = [], [], []
                for v in range(_GRP):
                    sl = pl.ds(base + v * 16, 16)
                    u = db[sl] - lo
                    m = (u >= 0) & (u < _TPB)
                    sel = lax.select(m, ones_i, zeros_i)
                    uu.append(u)
                    mm.append(m)
                    pp.append(plsc.cumsum(sel)[15])
                    plsc.addupdate_scatter(cntacc, [u], ones_f, mask=m)
                for v in range(_GRP):
                    sl = pl.ds(base + v * 16, 16)
                    psl = pl.ds(ptr, 16)
                    plsc.store_compressed(csrc.at[psl], sb[sl], mask=mm[v])
                    plsc.store_compressed(cdl.at[psl], uu[v], mask=mm[v])
                    plsc.store_compressed(ce.at[psl], eb[sl], mask=mm[v])
                    ptr = ptr + pp[v]
                do_flush = ptr >= _B2

                @pl.when(do_flush)
                def _():
                    flush(pend)
                    for j in range((_CAP - _B2) // 16):
                        s1 = pl.ds(j * 16, 16)
                        s2 = pl.ds(_B2 + j * 16, 16)
                        csrc[s1] = csrc[s2]
                        cdl[s1] = cdl[s2]
                        ce[s1] = ce[s2]

                return (jnp.where(do_flush, ptr - _B2, ptr),
                        jnp.where(do_flush, 1, pend))

            return lax.fori_loop(0, _SCC // (16 * _GRP), group_body, ptr)

        issue(0, dbuf0, sbuf0, ebuf0, semA)

        def pair_body(p, carry):
            ci0 = 2 * p
            issue(ci0 + 1, dbuf1, sbuf1, ebuf1, semB)
            drain(dbuf0, sbuf0, ebuf0, semA)
            carry = scan_chunk(dbuf0, sbuf0, ebuf0, carry)

            @pl.when(ci0 + 2 < _NSCC)
            def _():
                issue(ci0 + 2, dbuf0, sbuf0, ebuf0, semA)

            drain(dbuf1, sbuf1, ebuf1, semB)
            return scan_chunk(dbuf1, sbuf1, ebuf1, carry)

        ptr, pend = lax.fori_loop(0, _NSCC // 2, pair_body,
                                  (jnp.int32(0), jnp.int32(0)))

        @pl.when(pend > 0)
        def _():
            wait_gather()
            accumulate(sdl, sev)

        lane = lax.iota(jnp.int32, 16)
        for j in range(_B2 // 16):
            sl = pl.ds(j * 16, 16)
            keep = (lane + j * 16) < ptr
            cdl[sl] = jnp.where(keep, cdl[sl], _TRASH)
            ce[sl] = jnp.where(keep, ce[sl], 0.0)
            csrc[sl] = jnp.where(keep, csrc[sl], 0)
        fire_gather()
        wait_gather()
        accumulate(cdl, ce)

        pltpu.sync_copy(mxacc, mx_hbm.at[wid])
        pltpu.sync_copy(smacc, sm_hbm.at[wid])
        pltpu.sync_copy(cntacc, cnt_hbm.at[wid])

    return k(src, dst, e, hpack)


def _tc3_body(x_ref, mx_ref, sm_ref, cnt_ref,
              wself_ref, wneigh_ref, wneigh2_ref, wm0_ref, wm1_ref,
              b0_ref, bm0_ref, bm1_ref, out_ref):
    x = x_ref[...]
    mx = mx_ref[...]
    neigh = jnp.where(jnp.isfinite(mx), mx, 0.0)
    neigh2 = sm_ref[...] / jnp.maximum(cnt_ref[...], 1.0)
    rst = (jnp.dot(x, wself_ref[...], preferred_element_type=jnp.float32)
           + jnp.dot(neigh, wneigh_ref[...], preferred_element_type=jnp.float32)
           + jnp.dot(neigh2, wneigh2_ref[...], preferred_element_type=jnp.float32)
           + b0_ref[...])
    rst = rst + jnp.dot(_gelu(rst), wm0_ref[...],
                        preferred_element_type=jnp.float32) + bm0_ref[...]
    rst = rst + jnp.dot(_gelu(rst), wm1_ref[...],
                        preferred_element_type=jnp.float32) + bm1_ref[...]
    out_ref[...] = rst


def _tc3(x, mx, sm, cnt, wself, wneigh, wneigh2, wm0, wm1, b0, bm0, bm1):
    full = lambda shape: pl.BlockSpec(shape, lambda i: (0,) * len(shape))
    return pl.pallas_call(
        _tc3_body,
        grid=(_NB1,),
        in_specs=[
            pl.BlockSpec((_BN, D), lambda i: (i, 0)),
            pl.BlockSpec((_BN, D), lambda i: (i, 0)),
            pl.BlockSpec((_BN, D), lambda i: (i, 0)),
            pl.BlockSpec((_BN, 1), lambda i: (i, 0)),
            full((D, D)), full((D, D)), full((D, D)), full((D, D)), full((D, D)),
            full((1, D)), full((1, D)), full((1, D)),
        ],
        out_specs=pl.BlockSpec((_BN, D), lambda i: (i, 0)),
        out_shape=jax.ShapeDtypeStruct((N, D), jnp.float32),
    )(x, mx, sm, cnt, wself, wneigh, wneigh2, wm0, wm1, b0, bm0, bm1)


def kernel(x, edge_index, params):
    src = edge_index[0]
    dst = edge_index[1]

    wsp = (params['W_sub'] + params['W_src']).T
    wq = (params['W_dst'] - params['W_sub']).T
    wp = params['W_pool'].T
    wp2 = params['W_pool2'].T
    bp = params['b_pool'].reshape(1, D)
    bp2 = params['b_pool2'].reshape(1, D)
    s_tab, t_tab, hpack = _tc1(x, wsp, wq, wp, wp2, bp, bp2)

    xs, xd = _sc1(s_tab, t_tab, src, dst)

    wm = params['W_mul'].T
    ball = (params['b_sub'] + params['b_src'] + params['b_dst']
            + params['b_mul']).reshape(1, D)
    woutb = jnp.concatenate([params['W_out'][0], params['b_out']]).reshape(1, D + 1)
    e = _tc2(xs, xd, wm, ball, woutb)

    mx_h, sm_h, cnt_h = _sc2(src, dst, e, hpack)
    mx = mx_h[:, :_TPB, :].reshape(_NW * _TPB, D)[:N]
    sm = sm_h[:, :_TPB, :].reshape(_NW * _TPB, D)[:N]
    cnt = cnt_h[:, :_TPB].reshape(_NW * _TPB)[:N].reshape(N, 1)

    return _tc3(x, mx, sm, cnt,
                params['W_self'].T, params['W_neigh'].T, params['W_neigh2'].T,
                params['W_mlp0'].T, params['W_mlp1'].T,
                (params['b_self'] + params['b_neigh']
                 + params['b_neigh2']).reshape(1, D),
                params['b_mlp0'].reshape(1, D),
                params['b_mlp1'].reshape(1, D))


# speedup vs baseline: 1.0789x; 1.0198x over previous
import functools

import jax
import jax.numpy as jnp
from jax import lax
from jax.experimental import pallas as pl
from jax.experimental.pallas import tpu as pltpu
from jax.experimental.pallas import tpu_sc as plsc

N = 10000
E = 320000
D = 128

_NB1 = 10
_BN = N // _NB1
_NBE = 160
_BE = E // _NBE


def _gelu(x):
    return 0.5 * x * (1.0 + jax.lax.erf(x * 0.7071067811865476))


def _tc1_body(x_ref, wsp_ref, wq_ref, wp_ref, wp2_ref, bp_ref, bp2_ref,
              s_ref, t_ref, h_ref):
    x = x_ref[...]
    s_ref[:, :D] = x
    s_ref[:, D:] = jnp.dot(x, wsp_ref[...], preferred_element_type=jnp.float32)
    t_ref[:, :D] = x
    t_ref[:, D:] = jnp.dot(x, wq_ref[...], preferred_element_type=jnp.float32)
    h_ref[:, :D] = _gelu(jnp.dot(x, wp_ref[...], preferred_element_type=jnp.float32)
                         + bp_ref[...])
    h_ref[:, D:] = _gelu(jnp.dot(x, wp2_ref[...], preferred_element_type=jnp.float32)
                         + bp2_ref[...])


def _tc1(x, wsp, wq, wp, wp2, bp, bp2):
    full = lambda shape: pl.BlockSpec(shape, lambda i: (0,) * len(shape))
    return pl.pallas_call(
        _tc1_body,
        grid=(_NB1,),
        in_specs=[
            pl.BlockSpec((_BN, D), lambda i: (i, 0)),
            full((D, D)), full((D, D)), full((D, D)), full((D, D)),
            full((1, D)), full((1, D)),
        ],
        out_specs=[
            pl.BlockSpec((_BN, 2 * D), lambda i: (i, 0)),
            pl.BlockSpec((_BN, 2 * D), lambda i: (i, 0)),
            pl.BlockSpec((_BN, 2 * D), lambda i: (i, 0)),
        ],
        out_shape=[
            jax.ShapeDtypeStruct((N, 2 * D), jnp.float32),
            jax.ShapeDtypeStruct((N, 2 * D), jnp.float32),
            jax.ShapeDtypeStruct((N, 2 * D), jnp.float32),
        ],
    )(x, wsp, wq, wp, wp2, bp, bp2)


_NC, _NS = 2, 16
_NW = _NC * _NS
_GB = 128
_NCHUNK = E // _GB
_CPW = -(-_NCHUNK // _NW)


def _sc1(s_tab, t_tab, src, dst):
    mesh = plsc.VectorSubcoreMesh(core_axis_name="c", subcore_axis_name="s",
                                  num_cores=_NC, num_subcores=_NS)

    @functools.partial(
        pl.kernel,
        out_type=[jax.ShapeDtypeStruct((E, 2 * D), jnp.float32),
                  jax.ShapeDtypeStruct((E, 2 * D), jnp.float32)],
        mesh=mesh,
        scratch_types=[pltpu.VMEM((_GB,), jnp.int32),
                       pltpu.VMEM((_GB,), jnp.int32),
                       pltpu.VMEM((_GB,), jnp.int32),
                       pltpu.VMEM((_GB,), jnp.int32),
                       pltpu.VMEM((_GB, 2 * D), jnp.float32),
                       pltpu.VMEM((_GB, 2 * D), jnp.float32),
                       pltpu.SemaphoreType.DMA,
                       pltpu.SemaphoreType.DMA,
                       pltpu.SemaphoreType.DMA,
                       pltpu.SemaphoreType.DMA],
    )
    def k(s_hbm, t_hbm, src_hbm, dst_hbm, xs_hbm, xd_hbm,
          idx_s0, idx_d0, idx_s1, idx_d1, rows_s, rows_d,
          semi0, semi1, semgs, semgd):
        wid = lax.axis_index("s") * _NC + lax.axis_index("c")

        def issue_idx(c, i_s, i_d, sem):
            base = c * _GB
            pltpu.async_copy(src_hbm.at[pl.ds(base, _GB)], i_s, sem)
            pltpu.async_copy(dst_hbm.at[pl.ds(base, _GB)], i_d, sem)

        def wait_idx(i_s, i_d, sem):
            pltpu.make_async_copy(src_hbm.at[pl.ds(0, _GB)], i_s, sem).wait()
            pltpu.make_async_copy(src_hbm.at[pl.ds(0, _GB)], i_d, sem).wait()

        def half(jj, i_s, i_d, sem, i_s2, i_d2, sem2):
            c = wid + _NW * jj

            @pl.when(c < _NCHUNK)
            def _():
                base = c * _GB
                wait_idx(i_s, i_d, sem)
                pltpu.async_copy(s_hbm.at[i_s], rows_s, semgs)
                pltpu.async_copy(t_hbm.at[i_d], rows_d, semgd)
                c2 = wid + _NW * (jj + 1)

                @pl.when(c2 < _NCHUNK)
                def _():
                    issue_idx(c2, i_s2, i_d2, sem2)

                pltpu.make_async_copy(s_hbm.at[pl.ds(0, _GB)], rows_s,
                                      semgs).wait()
                pltpu.sync_copy(rows_s, xs_hbm.at[pl.ds(base, _GB)])
                pltpu.make_async_copy(s_hbm.at[pl.ds(0, _GB)], rows_d,
                                      semgd).wait()
                pltpu.sync_copy(rows_d, xd_hbm.at[pl.ds(base, _GB)])

        issue_idx(wid, idx_s0, idx_d0, semi0)

        def pair_body(p, carry):
            half(2 * p, idx_s0, idx_d0, semi0, idx_s1, idx_d1, semi1)
            half(2 * p + 1, idx_s1, idx_d1, semi1, idx_s0, idx_d0, semi0)
            return carry

        lax.fori_loop(0, (_CPW + 1) // 2, pair_body, 0)

    return k(s_tab, t_tab, src, dst)


def _tc2_body(xs_ref, xd_ref, wm_ref, ball_ref, wout_ref, e_ref):
    xs = xs_ref[...]
    xd = xd_ref[...]
    g = xs[:, :D] * xd[:, :D]
    z = (jnp.dot(g, wm_ref[...], preferred_element_type=jnp.float32)
         + xs[:, D:] + xd[:, D:] + ball_ref[...])
    ge = _gelu(z)
    s = jnp.sum(ge * wout_ref[0, :D], axis=1) + wout_ref[0, D]
    e_ref[0, 0, :] = jnp.where(s > 0, s, 0.2 * s)


def _tc2(xs, xd, wm, ball, woutb):
    full = lambda shape: pl.BlockSpec(shape, lambda i: (0,) * len(shape))
    e3 = pl.pallas_call(
        _tc2_body,
        grid=(_NBE,),
        in_specs=[
            pl.BlockSpec((_BE, 2 * D), lambda i: (i, 0)),
            pl.BlockSpec((_BE, 2 * D), lambda i: (i, 0)),
            full((D, D)), full((1, D)), full((1, D + 1)),
        ],
        out_specs=pl.BlockSpec((1, 1, _BE), lambda i: (i, 0, 0)),
        out_shape=jax.ShapeDtypeStruct((_NBE, 1, _BE), jnp.float32),
    )(xs, xd, wm, ball, woutb)
    return e3.reshape(E)


_TPB = 313
_ACC = 320
_TRASH = 313
_SCC = 2000
_NSCC = E // _SCC
_B2 = 128
_GRP = 5
_CAP = 224


def _sc2(src, dst, e, hpack):
    mesh = plsc.VectorSubcoreMesh(core_axis_name="c", subcore_axis_name="s",
                                  num_cores=_NC, num_subcores=_NS)

    @functools.partial(
        pl.kernel,
        out_type=[jax.ShapeDtypeStruct((_NW, _ACC, D), jnp.float32),
                  jax.ShapeDtypeStruct((_NW, _ACC, D), jnp.float32),
                  jax.ShapeDtypeStruct((_NW, _ACC), jnp.float32)],
        mesh=mesh,
        compiler_params=pltpu.CompilerParams(needs_layout_passes=False),
        scratch_types=[
            pltpu.VMEM((_SCC,), jnp.int32),
            pltpu.VMEM((_SCC,), jnp.int32),
            pltpu.VMEM((_SCC,), jnp.int32),
            pltpu.VMEM((_SCC,), jnp.int32),
            pltpu.VMEM((_SCC,), jnp.float32),
            pltpu.VMEM((_SCC,), jnp.float32),
            pltpu.VMEM((_CAP,), jnp.int32),
            pltpu.VMEM((_CAP,), jnp.int32),
            pltpu.VMEM((_CAP,), jnp.float32),
            pltpu.VMEM((_B2,), jnp.int32),
            pltpu.VMEM((_B2,), jnp.float32),
            pltpu.VMEM((_B2, 2 * D), jnp.float32),
            pltpu.VMEM((_ACC, D), jnp.float32),
            pltpu.VMEM((_ACC, D), jnp.float32),
            pltpu.VMEM((_ACC,), jnp.float32),
            pltpu.SemaphoreType.DMA,
            pltpu.SemaphoreType.DMA,
            pltpu.SemaphoreType.DMA,
        ],
    )
    def k(src_hbm, dst_hbm, e_hbm, h_hbm,
          mx_hbm, sm_hbm, cnt_hbm,
          dbuf0, dbuf1, sbuf0, sbuf1, ebuf0, ebuf1, csrc, cdl, ce,
          sdl, sev, grows, mxacc, smacc, cntacc, semA, semB, semg):
        wid = lax.axis_index("s") * _NC + lax.axis_index("c")
        lo = wid * _TPB

        neg = jnp.full((16,), -jnp.inf, jnp.float32)
        zero = jnp.zeros((16,), jnp.float32)
        ones_i = jnp.ones((16,), jnp.int32)
        zeros_i = jnp.zeros((16,), jnp.int32)
        ones_f = jnp.ones((16,), jnp.float32)

        def init_body(r, carry):
            for kk in range(D // 16):
                sl = pl.ds(kk * 16, 16)
                mxacc[r, sl] = neg
                smacc[r, sl] = zero
            return carry

        lax.fori_loop(0, _ACC, init_body, 0)
        for kk in range(_ACC // 16):
            cntacc[pl.ds(kk * 16, 16)] = zero

        def issue(ci, db, sb, eb, sem):
            base = ci * _SCC
            pltpu.async_copy(dst_hbm.at[pl.ds(base, _SCC)], db, sem)
            pltpu.async_copy(src_hbm.at[pl.ds(base, _SCC)], sb, sem)
            pltpu.async_copy(e_hbm.at[pl.ds(base, _SCC)], eb, sem)

        def drain(db, sb, eb, sem):
            pltpu.make_async_copy(dst_hbm.at[pl.ds(0, _SCC)], db, sem).wait()
            pltpu.make_async_copy(src_hbm.at[pl.ds(0, _SCC)], sb, sem).wait()
            pltpu.make_async_copy(e_hbm.at[pl.ds(0, _SCC)], eb, sem).wait()

        def fire_gather():
            pltpu.async_copy(h_hbm.at[csrc.at[pl.ds(0, _B2)]], grows, semg)

        def wait_gather():
            pltpu.make_async_copy(h_hbm.at[pl.ds(0, _B2)], grows, semg).wait()

        def accumulate(dlref, evref):
            def edge_grp(g, carry):
                base = g * 16
                dlv = dlref[pl.ds(base, 16)]
                evv_all = evref[pl.ds(base, 16)]
                for j in range(16):
                    i = base + j
                    dl = dlv[j]
                    evv = jnp.full((16,), evv_all[j], jnp.float32)
                    for kk in range(D // 16):
                        sl = pl.ds(kk * 16, 16)
                        sl2 = pl.ds(D + kk * 16, 16)
                        mxacc[dl, sl] = jnp.maximum(mxacc[dl, sl],
                                                    evv * grows[i, sl])
                        plsc.addupdate(smacc.at[dl, sl], evv * grows[i, sl2])
                return carry

            lax.fori_loop(0, _B2 // 16, edge_grp, 0)

        def flush(pend):
            @pl.when(pend > 0)
            def _():
                wait_gather()
                accumulate(sdl, sev)

            for j in range(_B2 // 16):
                sl = pl.ds(j * 16, 16)
                sdl[sl] = cdl[sl]
                sev[sl] = ce[sl]
            fire_gather()

        def scan_chunk(db, sb, eb, ptr):
            def group_body(g, carry):
                ptr, pend = carry
                base = g * (_GRP * 16)
                uu, mm, pp = [], [], []
                for v in range(_GRP):
                    sl = pl.ds(base + v * 16, 16)
                    u = db[sl] - lo
                    m = (u >= 0) & (u < _TPB)
                    sel = lax.select(m, ones_i, zeros_i)
                    uu.append(u)
                    mm.append(m)
                    pp.append(plsc.cumsum(sel)[15])
                    plsc.addupdate_scatter(cntacc, [u], ones_f, mask=m)
                for v in range(_GRP):
                    sl = pl.ds(base + v * 16, 16)
                    psl = pl.ds(ptr, 16)
                    plsc.store_compressed(csrc.at[psl], sb[sl], mask=mm[v])
                    plsc.store_compressed(cdl.at[psl], uu[v], mask=mm[v])
                    plsc.store_compressed(ce.at[psl], eb[sl], mask=mm[v])
                    ptr = ptr + pp[v]
                do_flush = ptr >= _B2

                @pl.when(do_flush)
                def _():
                    flush(pend)
                    for j in range((_CAP - _B2) // 16):
                        s1 = pl.ds(j * 16, 16)
                        s2 = pl.ds(_B2 + j * 16, 16)
                        csrc[s1] = csrc[s2]
                        cdl[s1] = cdl[s2]
                        ce[s1] = ce[s2]

                return (jnp.where(do_flush, ptr - _B2, ptr),
                        jnp.where(do_flush, 1, pend))

            return lax.fori_loop(0, _SCC // (16 * _GRP), group_body, ptr)

        issue(0, dbuf0, sbuf0, ebuf0, semA)

        def pair_body(p, carry):
            ci0 = 2 * p
            issue(ci0 + 1, dbuf1, sbuf1, ebuf1, semB)
            drain(dbuf0, sbuf0, ebuf0, semA)
            carry = scan_chunk(dbuf0, sbuf0, ebuf0, carry)

            @pl.when(ci0 + 2 < _NSCC)
            def _():
                issue(ci0 + 2, dbuf0, sbuf0, ebuf0, semA)

            drain(dbuf1, sbuf1, ebuf1, semB)
            return scan_chunk(dbuf1, sbuf1, ebuf1, carry)

        ptr, pend = lax.fori_loop(0, _NSCC // 2, pair_body,
                                  (jnp.int32(0), jnp.int32(0)))

        @pl.when(pend > 0)
        def _():
            wait_gather()
            accumulate(sdl, sev)

        lane = lax.iota(jnp.int32, 16)
        for j in range(_B2 // 16):
            sl = pl.ds(j * 16, 16)
            keep = (lane + j * 16) < ptr
            cdl[sl] = jnp.where(keep, cdl[sl], _TRASH)
            ce[sl] = jnp.where(keep, ce[sl], 0.0)
            csrc[sl] = jnp.where(keep, csrc[sl], 0)
        fire_gather()
        wait_gather()
        accumulate(cdl, ce)

        pltpu.sync_copy(mxacc, mx_hbm.at[wid])
        pltpu.sync_copy(smacc, sm_hbm.at[wid])
        pltpu.sync_copy(cntacc, cnt_hbm.at[wid])

    return k(src, dst, e, hpack)


def _tc3_body(x_ref, mx_ref, sm_ref, cnt_ref,
              wself_ref, wneigh_ref, wneigh2_ref, wm0_ref, wm1_ref,
              b0_ref, bm0_ref, bm1_ref, out_ref):
    x = x_ref[...]
    mx = mx_ref[...]
    neigh = jnp.where(jnp.isfinite(mx), mx, 0.0)
    neigh2 = sm_ref[...] / jnp.maximum(cnt_ref[...], 1.0)
    rst = (jnp.dot(x, wself_ref[...], preferred_element_type=jnp.float32)
           + jnp.dot(neigh, wneigh_ref[...], preferred_element_type=jnp.float32)
           + jnp.dot(neigh2, wneigh2_ref[...], preferred_element_type=jnp.float32)
           + b0_ref[...])
    rst = rst + jnp.dot(_gelu(rst), wm0_ref[...],
                        preferred_element_type=jnp.float32) + bm0_ref[...]
    rst = rst + jnp.dot(_gelu(rst), wm1_ref[...],
                        preferred_element_type=jnp.float32) + bm1_ref[...]
    out_ref[...] = rst


def _tc3(x, mx, sm, cnt, wself, wneigh, wneigh2, wm0, wm1, b0, bm0, bm1):
    full = lambda shape: pl.BlockSpec(shape, lambda i: (0,) * len(shape))
    return pl.pallas_call(
        _tc3_body,
        grid=(_NB1,),
        in_specs=[
            pl.BlockSpec((_BN, D), lambda i: (i, 0)),
            pl.BlockSpec((_BN, D), lambda i: (i, 0)),
            pl.BlockSpec((_BN, D), lambda i: (i, 0)),
            pl.BlockSpec((_BN, 1), lambda i: (i, 0)),
            full((D, D)), full((D, D)), full((D, D)), full((D, D)), full((D, D)),
            full((1, D)), full((1, D)), full((1, D)),
        ],
        out_specs=pl.BlockSpec((_BN, D), lambda i: (i, 0)),
        out_shape=jax.ShapeDtypeStruct((N, D), jnp.float32),
    )(x, mx, sm, cnt, wself, wneigh, wneigh2, wm0, wm1, b0, bm0, bm1)


def kernel(x, edge_index, params):
    src = edge_index[0]
    dst = edge_index[1]

    wsp = (params['W_sub'] + params['W_src']).T
    wq = (params['W_dst'] - params['W_sub']).T
    wp = params['W_pool'].T
    wp2 = params['W_pool2'].T
    bp = params['b_pool'].reshape(1, D)
    bp2 = params['b_pool2'].reshape(1, D)
    s_tab, t_tab, hpack = _tc1(x, wsp, wq, wp, wp2, bp, bp2)

    xs, xd = _sc1(s_tab, t_tab, src, dst)

    wm = params['W_mul'].T
    ball = (params['b_sub'] + params['b_src'] + params['b_dst']
            + params['b_mul']).reshape(1, D)
    woutb = jnp.concatenate([params['W_out'][0], params['b_out']]).reshape(1, D + 1)
    e = _tc2(xs, xd, wm, ball, woutb)

    mx_h, sm_h, cnt_h = _sc2(src, dst, e, hpack)
    mx = mx_h[:, :_TPB, :].reshape(_NW * _TPB, D)[:N]
    sm = sm_h[:, :_TPB, :].reshape(_NW * _TPB, D)[:N]
    cnt = cnt_h[:, :_TPB].reshape(_NW * _TPB)[:N].reshape(N, 1)

    return _tc3(x, mx, sm, cnt,
                params['W_self'].T, params['W_neigh'].T, params['W_neigh2'].T,
                params['W_mlp0'].T, params['W_mlp1'].T,
                (params['b_self'] + params['b_neigh']
                 + params['b_neigh2']).reshape(1, D),
                params['b_mlp0'].reshape(1, D),
                params['b_mlp1'].reshape(1, D))
